# Initial kernel scaffold; baseline (speedup 1.0000x reference)
import jax
import jax.numpy as jnp
from jax.experimental import pallas as pl


def kernel(features, edge_index, edge_weight, W1, W2, W3, fc_w, fc_b):
    raise NotImplementedError("write your pallas kernel here")


# speedup vs baseline: 4.8530x; 4.8530x over previous
import functools

import jax
import jax.numpy as jnp
from jax import lax
from jax.experimental import pallas as pl
from jax.experimental.pallas import tpu as pltpu
from jax.experimental.pallas import tpu_sc as plsc

NC = 2
NS = 16
LANES = 16
CH = 128


def _spmm_sc(feat, row1, col1, w1, zeros_tile):
    d = feat.shape[1]
    rpt = zeros_tile.shape[0]
    n_pad = rpt * NS
    e_pad = row1.shape[0]
    nch = e_pad // (NC * NS * CH)

    mesh = plsc.VectorSubcoreMesh(
        core_axis_name="c", subcore_axis_name="s", num_cores=NC,
        num_subcores=NS)

    @functools.partial(
        pl.kernel,
        mesh=mesh,
        out_type=jax.ShapeDtypeStruct((NC, n_pad, d), jnp.float32),
        scratch_types=[
            pltpu.VMEM((CH,), jnp.int32),
            pltpu.VMEM((CH,), jnp.int32),
            pltpu.VMEM((CH,), jnp.float32),
            pltpu.VMEM((CH, d), jnp.float32),
            pltpu.VMEM_SHARED((n_pad, d), jnp.float32),
            pltpu.SemaphoreType.DMA,
        ],
        compiler_params=pltpu.CompilerParams(use_tc_tiling_on_sc=False),
    )
    def spmm_kernel(feat_hbm, row_hbm, col_hbm, w_hbm, zero_hbm, out_hbm,
                    colv, rowv, wv, rows, acc, sem):
        cid = lax.axis_index("c")
        sid = lax.axis_index("s")
        wid = cid * NS + sid

        pltpu.sync_copy(zero_hbm, acc.at[pl.ds(sid * rpt, rpt)])
        plsc.subcore_barrier()

        def chunk_body(g, carry):
            base = (wid * nch + g) * CH
            pltpu.sync_copy(col_hbm.at[pl.ds(base, CH)], colv)
            pltpu.sync_copy(row_hbm.at[pl.ds(base, CH)], rowv)
            pltpu.sync_copy(w_hbm.at[pl.ds(base, CH)], wv)
            pltpu.async_copy(feat_hbm.at[colv], rows, sem).wait()

            @plsc.parallel_loop(0, CH // LANES, unroll=2)
            def mul_body(grp):
                wgrp = wv[pl.ds(grp * LANES, LANES)]
                for t in range(LANES):
                    w = wgrp[t]
                    e = grp * LANES + t
                    for j in range(d // LANES):
                        sl = pl.ds(j * LANES, LANES)
                        rows[e, sl] = rows[e, sl] * w

            pltpu.sync_copy(rows, acc.at[rowv], add=True)
            return carry

        lax.fori_loop(0, nch, chunk_body, 0)

        plsc.subcore_barrier()
        pltpu.sync_copy(acc.at[pl.ds(sid * rpt, rpt)],
                        out_hbm.at[cid, pl.ds(sid * rpt, rpt)])

    return spmm_kernel(feat, row1, col1, w1, zeros_tile)


def _dense_in_body(x_ref, w_ref, h1_ref, p_ref):
    m = jnp.dot(x_ref[...], w_ref[...], preferred_element_type=jnp.float32)
    h1_ref[...] = jnp.maximum(m[:, :64], 0.0)
    p_ref[...] = m[:, 64:]


def _combine_body(p_ref, h2_ref, t3_ref):
    s = p_ref[0] + p_ref[1]
    h2_ref[...] = jnp.maximum(s[:, :64], 0.0)
    t3_ref[...] = s[:, 64:]


def _final_body(h1_ref, h2_ref, q_ref, fcw_ref, fcb_ref, out_ref):
    h3 = jnp.maximum(q_ref[0] + q_ref[1], 0.0)
    a = jnp.concatenate([h1_ref[...], h2_ref[...], h3], axis=1)
    logits = jnp.dot(a, fcw_ref[...], preferred_element_type=jnp.float32)
    logits = logits + fcb_ref[...]
    ncls = 40
    colid = lax.broadcasted_iota(jnp.int32, logits.shape, 1)
    logits = jnp.where(colid < ncls, logits, -jnp.inf)
    m = jnp.max(logits, axis=1, keepdims=True)
    ex = jnp.exp(logits - m)
    s = jnp.sum(ex, axis=1, keepdims=True)
    out_ref[...] = logits - m - jnp.log(s)


def kernel(features, edge_index, edge_weight, W1, W2, W3, fc_w, fc_b):
    n, dfeat = features.shape
    e = edge_index.shape[1]
    d1 = W1.shape[1]
    d23 = W2.shape[1] + W3.shape[1]
    d3 = W3.shape[1]
    ncls = fc_w.shape[1]
    nw = NC * NS

    rpt = -(-n // (NS * 8)) * 8
    n_pad = rpt * NS

    nch = -(-e // (nw * CH))
    e_pad = nw * CH * nch
    row1 = jnp.pad(edge_index[0], (0, e_pad - e))
    col1 = jnp.pad(edge_index[1], (0, e_pad - e))
    w1 = jnp.pad(edge_weight, (0, e_pad - e))

    wcat = jnp.concatenate([W1, W2, W3], axis=1)

    blk = 2000
    grid = n // blk
    h1, p = pl.pallas_call(
        _dense_in_body,
        grid=(grid,),
        in_specs=[
            pl.BlockSpec((blk, dfeat), lambda i: (i, 0)),
            pl.BlockSpec((dfeat, d1 + d23), lambda i: (0, 0)),
        ],
        out_specs=[
            pl.BlockSpec((blk, d1), lambda i: (i, 0)),
            pl.BlockSpec((blk, d23), lambda i: (i, 0)),
        ],
        out_shape=[
            jax.ShapeDtypeStruct((n, d1), jnp.float32),
            jax.ShapeDtypeStruct((n, d23), jnp.float32),
        ],
    )(features, wcat)

    zeros128 = jnp.zeros((rpt, d23), jnp.float32)
    part1 = _spmm_sc(p, row1, col1, w1, zeros128)

    h2, t3 = pl.pallas_call(
        _combine_body,
        grid=(NS,),
        in_specs=[pl.BlockSpec((NC, rpt, d23), lambda i: (0, i, 0))],
        out_specs=[
            pl.BlockSpec((rpt, d1), lambda i: (i, 0)),
            pl.BlockSpec((rpt, d3), lambda i: (i, 0)),
        ],
        out_shape=[
            jax.ShapeDtypeStruct((n_pad, d1), jnp.float32),
            jax.ShapeDtypeStruct((n_pad, d3), jnp.float32),
        ],
    )(part1)

    zeros64 = jnp.zeros((rpt, d3), jnp.float32)
    part2 = _spmm_sc(t3, row1, col1, w1, zeros64)

    npad = 128
    fcw_pad = jnp.zeros((fc_w.shape[0], npad), jnp.float32).at[:, :ncls].set(fc_w)
    fcb_pad = jnp.zeros((1, npad), jnp.float32).at[0, :ncls].set(fc_b)
    out_pad = pl.pallas_call(
        _final_body,
        grid=(grid,),
        in_specs=[
            pl.BlockSpec((blk, d1), lambda i: (i, 0)),
            pl.BlockSpec((blk, d1), lambda i: (i, 0)),
            pl.BlockSpec((NC, blk, d3), lambda i: (0, i, 0)),
            pl.BlockSpec((fc_w.shape[0], npad), lambda i: (0, 0)),
            pl.BlockSpec((1, npad), lambda i: (0, 0)),
        ],
        out_specs=pl.BlockSpec((blk, npad), lambda i: (i, 0)),
        out_shape=jax.ShapeDtypeStruct((n, npad), jnp.float32),
    )(h1, h2, part2, fcw_pad, fcb_pad)
    return out_pad[:, :ncls]
